# Initial kernel scaffold; baseline (speedup 1.0000x reference)
import functools
import math

import jax
import jax.numpy as jnp
from jax import lax
from jax.experimental import pallas as pl
from jax.experimental.pallas import tpu as pltpu
from jax.experimental.pallas import tpu_sc as plsc

N_NODES_C = 10000
N_EDGES_C = 320000
D_FEAT_C = 128
INV_SQRT_D = 1.0 / math.sqrt(D_FEAT_C)

NC = 2
NS = 16
NW = NC * NS
LANES = 16

E_PER_TILE = N_EDGES_C // NW


def _rowsum_tc_kernel(x_ref, o_ref):
    o_ref[...] = jnp.sum(x_ref[...], axis=1, keepdims=True)


def _rowsum(node_embeds):
    n = node_embeds.shape[0]
    return pl.pallas_call(
        _rowsum_tc_kernel,
        out_shape=jax.ShapeDtypeStruct((n, 1), jnp.float32),
    )(node_embeds)


def _edge_score_sc(table_hbm, src_hbm, dst_hbm, out_hbm,
                   table_v, src_v, dst_v, out_v):
    wid = lax.axis_index("s") * NC + lax.axis_index("c")
    base = wid * E_PER_TILE
    pltpu.sync_copy(table_hbm, table_v)
    pltpu.sync_copy(src_hbm.at[pl.ds(base, E_PER_TILE)], src_v)
    pltpu.sync_copy(dst_hbm.at[pl.ds(base, E_PER_TILE)], dst_v)

    def body(i, carry):
        off = i * LANES
        si = src_v[pl.ds(off, LANES)]
        di = dst_v[pl.ds(off, LANES)]
        a = plsc.load_gather(table_v, [si])
        b = plsc.load_gather(table_v, [di])
        out_v[pl.ds(off, LANES)] = (a - b) * INV_SQRT_D
        return carry

    lax.fori_loop(0, E_PER_TILE // LANES, body, 0)
    pltpu.sync_copy(out_v, out_hbm.at[pl.ds(base, E_PER_TILE)])


@jax.jit
def kernel(node_embeds, edge_index):
    rowsum = _rowsum(node_embeds).reshape(N_NODES_C)
    idx = edge_index.astype(jnp.int32)
    src = idx[0]
    dst = idx[1]

    mesh = plsc.VectorSubcoreMesh(core_axis_name="c", subcore_axis_name="s")
    score = pl.kernel(
        _edge_score_sc,
        out_type=jax.ShapeDtypeStruct((N_EDGES_C,), jnp.float32),
        mesh=mesh,
        scratch_types=[
            pltpu.VMEM((N_NODES_C,), jnp.float32),
            pltpu.VMEM((E_PER_TILE,), jnp.int32),
            pltpu.VMEM((E_PER_TILE,), jnp.int32),
            pltpu.VMEM((E_PER_TILE,), jnp.float32),
        ],
    )(rowsum, src, dst)
    return score


# speedup vs baseline: 32.9203x; 32.9203x over previous
import functools
import math

import jax
import jax.numpy as jnp
from jax import lax
from jax.experimental import pallas as pl
from jax.experimental.pallas import tpu as pltpu
from jax.experimental.pallas import tpu_sc as plsc

N_NODES_C = 10000
N_EDGES_C = 320000
D_FEAT_C = 128
INV_SQRT_D = 1.0 / math.sqrt(D_FEAT_C)

NC = 2
NS = 16
NW = NC * NS
LANES = 16

E_PER_TILE = N_EDGES_C // NW


def _rowsum_tc_kernel(x_ref, o_ref):
    o_ref[...] = jnp.sum(x_ref[...], axis=1, keepdims=True)


def _rowsum(node_embeds):
    n = node_embeds.shape[0]
    return pl.pallas_call(
        _rowsum_tc_kernel,
        out_shape=jax.ShapeDtypeStruct((n, 1), jnp.float32),
    )(node_embeds)


def _edge_score_sc(table_hbm, src_hbm, dst_hbm, out_hbm,
                   table_v, src_v, dst_v, out_v):
    wid = lax.axis_index("s") * NC + lax.axis_index("c")
    base = wid * E_PER_TILE
    pltpu.sync_copy(table_hbm, table_v)
    pltpu.sync_copy(src_hbm.at[pl.ds(base, E_PER_TILE)], src_v)
    pltpu.sync_copy(dst_hbm.at[pl.ds(base, E_PER_TILE)], dst_v)

    def body(i, carry):
        off = i * LANES
        si = src_v[pl.ds(off, LANES)]
        di = dst_v[pl.ds(off, LANES)]
        a = plsc.load_gather(table_v, [si])
        b = plsc.load_gather(table_v, [di])
        out_v[pl.ds(off, LANES)] = (a - b) * INV_SQRT_D
        return carry

    lax.fori_loop(0, E_PER_TILE // LANES, body, 0)
    pltpu.sync_copy(out_v, out_hbm.at[pl.ds(base, E_PER_TILE)])


@jax.jit
def kernel(node_embeds, edge_index):
    rowsum = _rowsum(node_embeds).reshape(N_NODES_C)
    idx = edge_index.astype(jnp.int32)
    src = idx[0]
    dst = idx[1]

    mesh = plsc.VectorSubcoreMesh(core_axis_name="c", subcore_axis_name="s")
    score = pl.kernel(
        _edge_score_sc,
        out_type=jax.ShapeDtypeStruct((N_EDGES_C,), jnp.float32),
        mesh=mesh,
        scratch_types=[
            pltpu.VMEM((N_NODES_C,), jnp.float32),
            pltpu.VMEM((E_PER_TILE,), jnp.int32),
            pltpu.VMEM((E_PER_TILE,), jnp.int32),
            pltpu.VMEM((E_PER_TILE,), jnp.float32),
        ],
        compiler_params=pltpu.CompilerParams(needs_layout_passes=False),
    )(rowsum, src, dst)
    return score


# speedup vs baseline: 35.8620x; 1.0894x over previous
import functools
import math

import jax
import jax.numpy as jnp
from jax import lax
from jax.experimental import pallas as pl
from jax.experimental.pallas import tpu as pltpu
from jax.experimental.pallas import tpu_sc as plsc

N_NODES_C = 10000
N_EDGES_C = 320000
D_FEAT_C = 128
INV_SQRT_D = 1.0 / math.sqrt(D_FEAT_C)

NC = 2
NS = 16
NW = NC * NS
LANES = 16

E_PER_TILE = N_EDGES_C // NW


def _rowsum_tc_kernel(x_ref, o_ref):
    o_ref[...] = jnp.sum(x_ref[...], axis=1, keepdims=True)


def _rowsum(node_embeds):
    n = node_embeds.shape[0]
    return pl.pallas_call(
        _rowsum_tc_kernel,
        out_shape=jax.ShapeDtypeStruct((n, 1), jnp.float32),
    )(node_embeds)


def _edge_score_sc(table_hbm, src_hbm, dst_hbm, out_hbm,
                   table_v, src_v, dst_v, out_v, sem):
    wid = lax.axis_index("s") * NC + lax.axis_index("c")
    base = wid * E_PER_TILE
    ct = pltpu.make_async_copy(table_hbm, table_v, sem)
    cs = pltpu.make_async_copy(src_hbm.at[pl.ds(base, E_PER_TILE)], src_v, sem)
    cd = pltpu.make_async_copy(dst_hbm.at[pl.ds(base, E_PER_TILE)], dst_v, sem)
    ct.start()
    cs.start()
    cd.start()
    ct.wait()
    cs.wait()
    cd.wait()

    @plsc.parallel_loop(0, E_PER_TILE // LANES, step=1, unroll=8)
    def body(i):
        off = i * LANES
        si = src_v[pl.ds(off, LANES)]
        di = dst_v[pl.ds(off, LANES)]
        a = plsc.load_gather(table_v, [si])
        b = plsc.load_gather(table_v, [di])
        out_v[pl.ds(off, LANES)] = (a - b) * INV_SQRT_D

    pltpu.sync_copy(out_v, out_hbm.at[pl.ds(base, E_PER_TILE)])


@jax.jit
def kernel(node_embeds, edge_index):
    rowsum = _rowsum(node_embeds).reshape(N_NODES_C)
    idx = edge_index.astype(jnp.int32)
    src = idx[0]
    dst = idx[1]

    mesh = plsc.VectorSubcoreMesh(core_axis_name="c", subcore_axis_name="s")
    score = pl.kernel(
        _edge_score_sc,
        out_type=jax.ShapeDtypeStruct((N_EDGES_C,), jnp.float32),
        mesh=mesh,
        scratch_types=[
            pltpu.VMEM((N_NODES_C,), jnp.float32),
            pltpu.VMEM((E_PER_TILE,), jnp.int32),
            pltpu.VMEM((E_PER_TILE,), jnp.int32),
            pltpu.VMEM((E_PER_TILE,), jnp.float32),
            pltpu.SemaphoreType.DMA,
        ],
        compiler_params=pltpu.CompilerParams(needs_layout_passes=False),
    )(rowsum, src, dst)
    return score
